# Initial kernel scaffold; baseline (speedup 1.0000x reference)
import jax
import jax.numpy as jnp
from jax.experimental import pallas as pl


def kernel(query_len, key_len, relative_attention_bias):
    raise NotImplementedError("write your pallas kernel here")


# speedup vs baseline: 109.5915x; 109.5915x over previous
import math

import jax
import jax.numpy as jnp
from jax.experimental import pallas as pl
from jax.experimental.pallas import tpu as pltpu

_NUM_BUCKETS = 32
_NUM_HEADS = 16
_Q = 4096
_K = 4096
_D = 8192
_BQ = 8


def _bias_kernel(shift_ref, table_ref, out_ref, diag_ref, w_ref):
    i = pl.program_id(0)

    @pl.when(i == 0)
    def _compute_diag():
        shift = shift_ref[0]
        t = jax.lax.broadcasted_iota(jnp.int32, (1, _D), 1)
        rel = t - (_Q - 1) + shift
        n = -rel
        half = _NUM_BUCKETS // 2
        neg = jnp.where(n < 0, half, 0)
        n = jnp.abs(n)
        max_exact = half // 2
        scale = (half - max_exact) / math.log(128 / max_exact)
        log_val = (jnp.log(n.astype(jnp.float32) / max_exact + 1e-10)
                   * scale).astype(jnp.int32)
        bucket = jnp.where(n < max_exact, n, max_exact + log_val)
        bucket = jnp.clip(bucket, 0, half - 1) + neg
        onehot = (jax.lax.broadcasted_iota(jnp.int32, (_NUM_BUCKETS, _D), 0)
                  == bucket).astype(jnp.float32)
        diag_ref[...] = jax.lax.dot_general(
            table_ref[...], onehot, (((0,), (0,)), ((), ())),
            preferred_element_type=jnp.float32)

    start = _Q - (i + 1) * _BQ
    w_ref[...] = pltpu.roll(diag_ref[...], -start, axis=1)
    for r in range(_BQ):
        off = _BQ - 1 - r
        out_ref[:, r, :] = w_ref[:, off:off + _K]


def kernel(query_len, key_len, relative_attention_bias):
    shift = jnp.asarray(key_len - query_len, jnp.int32).reshape(1)
    out = pl.pallas_call(
        _bias_kernel,
        grid_spec=pltpu.PrefetchScalarGridSpec(
            num_scalar_prefetch=1,
            grid=(_Q // _BQ,),
            in_specs=[
                pl.BlockSpec((_NUM_BUCKETS, _NUM_HEADS), lambda i, s: (0, 0)),
            ],
            out_specs=pl.BlockSpec((_NUM_HEADS, _BQ, _K), lambda i, s: (0, i, 0)),
            scratch_shapes=[pltpu.VMEM((_NUM_HEADS, _D), jnp.float32),
                            pltpu.VMEM((_NUM_HEADS, _D), jnp.float32)],
        ),
        out_shape=jax.ShapeDtypeStruct((_NUM_HEADS, _Q, _K), jnp.float32),
        compiler_params=pltpu.CompilerParams(
            dimension_semantics=("arbitrary",),
        ),
    )(shift, relative_attention_bias)
    return out[None]


# speedup vs baseline: 157.6996x; 1.4390x over previous
import math

import jax
import jax.numpy as jnp
from jax.experimental import pallas as pl
from jax.experimental.pallas import tpu as pltpu

_NUM_BUCKETS = 32
_NUM_HEADS = 16
_Q = 4096
_K = 4096
_D = 8192
_BQ = 32


def _bias_kernel(shift_ref, table_ref, out_ref, diag_ref, w_ref):
    i = pl.program_id(0)

    @pl.when(i == 0)
    def _compute_diag():
        shift = shift_ref[0]
        t = jax.lax.broadcasted_iota(jnp.int32, (1, _D), 1)
        rel = t - (_Q - 1) + shift
        n = -rel
        half = _NUM_BUCKETS // 2
        neg = jnp.where(n < 0, half, 0)
        n = jnp.abs(n)
        max_exact = half // 2
        scale = (half - max_exact) / math.log(128 / max_exact)
        log_val = (jnp.log(n.astype(jnp.float32) / max_exact + 1e-10)
                   * scale).astype(jnp.int32)
        bucket = jnp.where(n < max_exact, n, max_exact + log_val)
        bucket = jnp.clip(bucket, 0, half - 1) + neg
        onehot = (jax.lax.broadcasted_iota(jnp.int32, (_NUM_BUCKETS, _D), 0)
                  == bucket).astype(jnp.float32)
        diag_ref[...] = jax.lax.dot_general(
            table_ref[...], onehot, (((0,), (0,)), ((), ())),
            preferred_element_type=jnp.float32)

    start = _Q - (i + 1) * _BQ
    w_ref[...] = pltpu.roll(diag_ref[...], -start, axis=1)
    for r in range(_BQ):
        off = _BQ - 1 - r
        out_ref[:, r, :] = w_ref[:, off:off + _K]


def kernel(query_len, key_len, relative_attention_bias):
    shift = jnp.asarray(key_len - query_len, jnp.int32).reshape(1)
    out = pl.pallas_call(
        _bias_kernel,
        grid_spec=pltpu.PrefetchScalarGridSpec(
            num_scalar_prefetch=1,
            grid=(_Q // _BQ,),
            in_specs=[
                pl.BlockSpec((_NUM_BUCKETS, _NUM_HEADS), lambda i, s: (0, 0)),
            ],
            out_specs=pl.BlockSpec((_NUM_HEADS, _BQ, _K), lambda i, s: (0, i, 0)),
            scratch_shapes=[pltpu.VMEM((_NUM_HEADS, _D), jnp.float32),
                            pltpu.VMEM((_NUM_HEADS, _D), jnp.float32)],
        ),
        out_shape=jax.ShapeDtypeStruct((_NUM_HEADS, _Q, _K), jnp.float32),
        compiler_params=pltpu.CompilerParams(
            dimension_semantics=("arbitrary",),
        ),
    )(shift, relative_attention_bias)
    return out[None]


# speedup vs baseline: 166.1649x; 1.0537x over previous
import math

import jax
import jax.numpy as jnp
from jax.experimental import pallas as pl
from jax.experimental.pallas import tpu as pltpu

_NUM_BUCKETS = 32
_NUM_HEADS = 16
_Q = 4096
_K = 4096
_D = 8192
_BQ = 64


def _bias_kernel(shift_ref, table_ref, out_ref, diag_ref, w_ref):
    i = pl.program_id(0)

    @pl.when(i == 0)
    def _compute_diag():
        shift = shift_ref[0]
        t = jax.lax.broadcasted_iota(jnp.int32, (1, _D), 1)
        rel = t - (_Q - 1) + shift
        n = -rel
        half = _NUM_BUCKETS // 2
        neg = jnp.where(n < 0, half, 0)
        n = jnp.abs(n)
        max_exact = half // 2
        scale = (half - max_exact) / math.log(128 / max_exact)
        log_val = (jnp.log(n.astype(jnp.float32) / max_exact + 1e-10)
                   * scale).astype(jnp.int32)
        bucket = jnp.where(n < max_exact, n, max_exact + log_val)
        bucket = jnp.clip(bucket, 0, half - 1) + neg
        onehot = (jax.lax.broadcasted_iota(jnp.int32, (_NUM_BUCKETS, _D), 0)
                  == bucket).astype(jnp.float32)
        diag_ref[...] = jax.lax.dot_general(
            table_ref[...], onehot, (((0,), (0,)), ((), ())),
            preferred_element_type=jnp.float32)

    start = _Q - (i + 1) * _BQ
    w_ref[...] = pltpu.roll(diag_ref[...], -start, axis=1)
    for r in range(_BQ):
        off = _BQ - 1 - r
        out_ref[:, r, :] = w_ref[:, off:off + _K]


def kernel(query_len, key_len, relative_attention_bias):
    shift = jnp.asarray(key_len - query_len, jnp.int32).reshape(1)
    out = pl.pallas_call(
        _bias_kernel,
        grid_spec=pltpu.PrefetchScalarGridSpec(
            num_scalar_prefetch=1,
            grid=(_Q // _BQ,),
            in_specs=[
                pl.BlockSpec((_NUM_BUCKETS, _NUM_HEADS), lambda i, s: (0, 0)),
            ],
            out_specs=pl.BlockSpec((_NUM_HEADS, _BQ, _K), lambda i, s: (0, i, 0)),
            scratch_shapes=[pltpu.VMEM((_NUM_HEADS, _D), jnp.float32),
                            pltpu.VMEM((_NUM_HEADS, _D), jnp.float32)],
        ),
        out_shape=jax.ShapeDtypeStruct((_NUM_HEADS, _Q, _K), jnp.float32),
        compiler_params=pltpu.CompilerParams(
            dimension_semantics=("arbitrary",),
        ),
    )(shift, relative_attention_bias)
    return out[None]
